# speedup vs baseline: 1.6246x; 1.6246x over previous
import functools

import jax
import jax.numpy as jnp
from jax import lax
from jax.experimental import pallas as pl
from jax.experimental.pallas import tpu as pltpu
from jax.experimental.pallas import tpu_sc as plsc

VOCAB = 4096
D = 4096
NC, NS = 2, 16
NW = NC * NS
B = 4 * 2048
B_PER_W = B // NW
CHUNK = 16
N_CHUNKS = B_PER_W // CHUNK

_mesh = plsc.VectorSubcoreMesh(
    core_axis_name="c", subcore_axis_name="s", num_cores=NC, num_subcores=NS
)


@functools.partial(
    pl.kernel,
    out_type=jax.ShapeDtypeStruct((B, D), jnp.float32),
    mesh=_mesh,
    scratch_types=[
        pltpu.VMEM((B_PER_W,), jnp.int32),
        pltpu.VMEM((CHUNK, D), jnp.float32),
        pltpu.SemaphoreType.DMA,
    ],
)
def _gather_rows(idx_hbm, table_hbm, out_hbm, idx_v, rows_v, sem):
    wid = lax.axis_index("s") * NC + lax.axis_index("c")
    base = wid * B_PER_W
    pltpu.sync_copy(idx_hbm.at[pl.ds(base, B_PER_W)], idx_v)

    @pl.loop(0, N_CHUNKS)
    def _chunk(j):
        row0 = j * CHUNK
        pltpu.async_copy(
            table_hbm.at[idx_v.at[pl.ds(row0, CHUNK)]], rows_v, sem
        ).wait()
        pltpu.sync_copy(rows_v, out_hbm.at[pl.ds(base + row0, CHUNK)])


def kernel(idx, embedding):
    flat = idx.reshape(-1).astype(jnp.int32)
    out = _gather_rows(flat, embedding)
    return out.reshape(idx.shape + (VOCAB,))


# speedup vs baseline: 1.7792x; 1.0952x over previous
import functools

import jax
import jax.numpy as jnp
from jax import lax
from jax.experimental import pallas as pl
from jax.experimental.pallas import tpu as pltpu
from jax.experimental.pallas import tpu_sc as plsc

VOCAB = 4096
D = 4096
NC, NS = 2, 16
NW = NC * NS
B = 4 * 2048
B_PER_W = B // NW
CHUNK = 8
N_CHUNKS = B_PER_W // CHUNK

_mesh = plsc.VectorSubcoreMesh(
    core_axis_name="c", subcore_axis_name="s", num_cores=NC, num_subcores=NS
)


@functools.partial(
    pl.kernel,
    out_type=jax.ShapeDtypeStruct((B, D), jnp.float32),
    mesh=_mesh,
    scratch_types=[
        pltpu.VMEM((B_PER_W,), jnp.int32),
        pltpu.VMEM((CHUNK, D), jnp.float32),
        pltpu.VMEM((CHUNK, D), jnp.float32),
        pltpu.SemaphoreType.DMA,
        pltpu.SemaphoreType.DMA,
        pltpu.SemaphoreType.DMA,
        pltpu.SemaphoreType.DMA,
    ],
)
def _gather_rows(idx_hbm, table_hbm, out_hbm, idx_v, rows0, rows1,
                 g0, g1, w0, w1):
    wid = lax.axis_index("s") * NC + lax.axis_index("c")
    base = wid * B_PER_W
    pltpu.sync_copy(idx_hbm.at[pl.ds(base, B_PER_W)], idx_v)

    bufs, gsems, wsems = (rows0, rows1), (g0, g1), (w0, w1)

    def gather_desc(j, b):
        return pltpu.make_async_copy(
            table_hbm.at[idx_v.at[pl.ds(j * CHUNK, CHUNK)]], bufs[b], gsems[b]
        )

    def write_desc(j, b):
        return pltpu.make_async_copy(
            bufs[b], out_hbm.at[pl.ds(base + j * CHUNK, CHUNK)], wsems[b]
        )

    gather_desc(0, 0).start()
    gather_desc(1, 1).start()

    @pl.loop(0, N_CHUNKS - 2, step=2)
    def _pair(i):
        for b in range(2):
            j = i + b
            gather_desc(j, b).wait()
            write_desc(j, b).start()
            write_desc(j, b).wait()
            gather_desc(j + 2, b).start()

    for b in range(2):
        j = N_CHUNKS - 2 + b
        gather_desc(j, b).wait()
        write_desc(j, b).start()
        write_desc(j, b).wait()


def kernel(idx, embedding):
    flat = idx.reshape(-1).astype(jnp.int32)
    out = _gather_rows(flat, embedding)
    return out.reshape(idx.shape + (VOCAB,))
